# Initial kernel scaffold; baseline (speedup 1.0000x reference)
import jax
import jax.numpy as jnp
from jax.experimental import pallas as pl


def kernel(init_position, time_vector, node_type, edge_index, edge_features, params):
    raise NotImplementedError("write your pallas kernel here")


# speedup vs baseline: 2.6678x; 2.6678x over previous
import functools

import jax
import jax.numpy as jnp
from jax import lax
from jax.experimental import pallas as pl
from jax.experimental.pallas import tpu as pltpu
from jax.experimental.pallas import tpu_sc as plsc

_N = 10000
_E = 160000
_D = 128
_NW = 32
_CH = 128
_NCHUNK = _E // _CH
_CPW = _NCHUNK // _NW
_REM = _NCHUNK - _CPW * _NW
_ZR = 16
_NG = _N // _ZR


def _sc_mesh():
    return plsc.VectorSubcoreMesh(core_axis_name="c", subcore_axis_name="s")


def _gather_pairs(x, src, dst):
    @functools.partial(
        pl.kernel,
        out_type=(jax.ShapeDtypeStruct((_E, _D), jnp.float32),
                  jax.ShapeDtypeStruct((_E, _D), jnp.float32)),
        mesh=_sc_mesh(),
        scratch_types=[
            pltpu.VMEM((_CH,), jnp.int32),
            pltpu.VMEM((_CH, _D), jnp.float32),
            pltpu.VMEM((_CH,), jnp.int32),
            pltpu.VMEM((_CH, _D), jnp.float32),
            pltpu.SemaphoreType.DMA,
        ],
    )
    def k(x_hbm, src_hbm, dst_hbm, xs_hbm, xd_hbm, idx_a, rows_a, idx_b, rows_b, sem):
        cid = lax.axis_index("c")
        sid = lax.axis_index("s")
        wid = sid * 2 + cid
        nch = jnp.where(wid < _REM, _CPW + 1, _CPW)

        def body(t, carry):
            c = wid + t * _NW
            off = pl.multiple_of(c * _CH, 8)
            pltpu.sync_copy(src_hbm.at[pl.ds(off, _CH)], idx_a)
            pltpu.sync_copy(dst_hbm.at[pl.ds(off, _CH)], idx_b)
            ga = pltpu.async_copy(x_hbm.at[idx_a], rows_a, sem)
            gb = pltpu.async_copy(x_hbm.at[idx_b], rows_b, sem)
            ga.wait()
            gb.wait()
            pltpu.sync_copy(rows_a, xs_hbm.at[pl.ds(off, _CH)])
            pltpu.sync_copy(rows_b, xd_hbm.at[pl.ds(off, _CH)])
            return carry

        lax.fori_loop(0, nch, body, 0)

    return k(x, src, dst)


def _segment_partials(e, dst):
    @functools.partial(
        pl.kernel,
        out_type=jax.ShapeDtypeStruct((2, _N, _D), jnp.float32),
        mesh=_sc_mesh(),
        scratch_types=[
            pltpu.VMEM((_CH,), jnp.int32),
            pltpu.VMEM((_CH, _D), jnp.float32),
            pltpu.VMEM((_ZR, _D), jnp.float32),
            pltpu.VMEM_SHARED((_N, _D), jnp.float32),
        ],
    )
    def k(e_hbm, dst_hbm, out_hbm, idx_v, rows_v, zbuf, acc):
        cid = lax.axis_index("c")
        sid = lax.axis_index("s")
        wid = sid * 2 + cid

        zero = jnp.zeros((16,), jnp.float32)
        for r in range(_ZR):
            for l in range(_D // 16):
                zbuf[r, pl.ds(16 * l, 16)] = zero
        ngrp = jnp.where(sid < _NG - 16 * (_NG // 16), _NG // 16 + 1, _NG // 16)

        def zfill(gk, carry):
            g = sid + gk * 16
            roff = pl.multiple_of(g * _ZR, 8)
            pltpu.sync_copy(zbuf, acc.at[pl.ds(roff, _ZR)])
            return carry

        lax.fori_loop(0, ngrp, zfill, 0)
        plsc.subcore_barrier()

        nch = jnp.where(wid < _REM, _CPW + 1, _CPW)

        def body(t, carry):
            c = wid + t * _NW
            off = pl.multiple_of(c * _CH, 8)
            pltpu.sync_copy(dst_hbm.at[pl.ds(off, _CH)], idx_v)
            pltpu.sync_copy(e_hbm.at[pl.ds(off, _CH)], rows_v)
            pltpu.sync_copy(rows_v, acc.at[idx_v], add=True)
            return carry

        lax.fori_loop(0, nch, body, 0)
        plsc.subcore_barrier()

        def wback(gk, carry):
            g = sid + gk * 16
            roff = pl.multiple_of(g * _ZR, 8)
            pltpu.sync_copy(acc.at[pl.ds(roff, _ZR)],
                            out_hbm.at[cid, pl.ds(roff, _ZR)])
            return carry

        lax.fori_loop(0, ngrp, wback, 0)

    return k(e, dst)


def _mlp_pallas(groups, w1s, b1, w2, b2, w3, b3, ln_g, ln_b, resid_gi, br):
    flat = [a for g in groups for a in g]
    sizes = [len(g) for g in groups]
    n = flat[0].shape[0]
    num_in = len(flat)
    ng = len(w1s)
    has_ln = ln_g is not None

    row_spec = lambda d: pl.BlockSpec((br, d), lambda i: (i, 0))
    full_spec = lambda s: pl.BlockSpec(s, lambda i: (0, 0))

    in_specs = [row_spec(a.shape[1]) for a in flat]
    in_specs += [full_spec(w.shape) for w in w1s]
    b1r = b1.reshape(1, _D)
    b2r = b2.reshape(1, _D)
    b3r = b3.reshape(1, _D)
    extras = [b1r, w2, b2r, w3, b3r]
    in_specs += [full_spec(b1r.shape), full_spec(w2.shape),
                 full_spec(b2r.shape), full_spec(w3.shape),
                 full_spec(b3r.shape)]
    if has_ln:
        extras += [ln_g.reshape(1, _D), ln_b.reshape(1, _D)]
        in_specs += [full_spec((1, _D)), full_spec((1, _D))]

    def body(*refs):
        irefs = refs[:num_in]
        wrefs = refs[num_in:num_in + ng]
        rest = refs[num_in + ng:]
        b1_r, w2_r, b2_r, w3_r, b3_r = rest[:5]
        out_ref = refs[-1]
        pos = 0
        h = None
        res = None
        for gi, sz in enumerate(sizes):
            xg = irefs[pos][...]
            for j in range(1, sz):
                xg = xg + irefs[pos + j][...]
            t = jnp.dot(xg, wrefs[gi][...], preferred_element_type=jnp.float32)
            h = t if h is None else h + t
            if resid_gi is not None and gi == resid_gi:
                res = xg
            pos += sz
        h = jax.nn.relu(h + b1_r[...])
        h = jax.nn.relu(jnp.dot(h, w2_r[...], preferred_element_type=jnp.float32) + b2_r[...])
        h = jnp.dot(h, w3_r[...], preferred_element_type=jnp.float32) + b3_r[...]
        if has_ln:
            g_r, bb_r = rest[5], rest[6]
            mu = jnp.mean(h, axis=-1, keepdims=True)
            var = jnp.mean((h - mu) * (h - mu), axis=-1, keepdims=True)
            h = (h - mu) * lax.rsqrt(var + 1e-5) * g_r[...] + bb_r[...]
        if res is not None:
            h = res + h
        out_ref[...] = h

    return pl.pallas_call(
        body,
        grid=(n // br,),
        in_specs=in_specs,
        out_specs=pl.BlockSpec((br, _D), lambda i: (i, 0)),
        out_shape=jax.ShapeDtypeStruct((n, _D), jnp.float32),
    )(*flat, *w1s, *extras)


def kernel(init_position, time_vector, node_type, edge_index, edge_features, params):
    p = params
    onehot = jax.nn.one_hot(node_type, 9, dtype=jnp.float32)
    nf = jnp.concatenate(
        [init_position, time_vector[:, None], onehot,
         jnp.zeros((_N, 4), jnp.float32)], axis=1)
    ef = jnp.concatenate(
        [edge_features, jnp.zeros((_E, 5), jnp.float32)], axis=1)
    src = edge_index[0]
    dst = edge_index[1]

    ne = p["node_enc"]
    w1n = jnp.concatenate([ne["W"][0], jnp.zeros((4, _D), jnp.float32)], axis=0)
    x = _mlp_pallas([[nf]], [w1n], ne["b"][0], ne["W"][1], ne["b"][1],
                    ne["W"][2], ne["b"][2], ne["ln_g"], ne["ln_b"],
                    resid_gi=None, br=1000)

    ee = p["edge_enc"]
    w1e = jnp.concatenate([ee["W"][0], jnp.zeros((5, _D), jnp.float32)], axis=0)
    e = _mlp_pallas([[ef]], [w1e], ee["b"][0], ee["W"][1], ee["b"][1],
                    ee["W"][2], ee["b"][2], ee["ln_g"], ee["ln_b"],
                    resid_gi=None, br=1000)

    for sp in p["proc"]:
        xs, xd = _gather_pairs(x, src, dst)
        ew = sp["edge"]
        wa = ew["W"][0][:_D]
        wb = ew["W"][0][_D:2 * _D]
        wc = ew["W"][0][2 * _D:]
        e = _mlp_pallas([[xs], [xd], [e]], [wa, wb, wc], ew["b"][0],
                        ew["W"][1], ew["b"][1], ew["W"][2], ew["b"][2],
                        ew["ln_g"], ew["ln_b"], resid_gi=2, br=1000)
        parts = _segment_partials(e, dst)
        nw = sp["node"]
        na = nw["W"][0][:_D]
        nb = nw["W"][0][_D:]
        x = _mlp_pallas([[x], [parts[0], parts[1]]], [na, nb], nw["b"][0],
                        nw["W"][1], nw["b"][1], nw["W"][2], nw["b"][2],
                        nw["ln_g"], nw["ln_b"], resid_gi=0, br=1000)

    dp = p["dec"]
    w3d = jnp.pad(dp["W"][2], ((0, 0), (0, _D - 2)))
    b3d = jnp.pad(dp["b"][2], (0, _D - 2))
    out = _mlp_pallas([[x]], [dp["W"][0]], dp["b"][0], dp["W"][1], dp["b"][1],
                      w3d, b3d, None, None, resid_gi=None, br=1000)
    return init_position + out[:, :2]


# speedup vs baseline: 3.0167x; 1.1308x over previous
import functools

import jax
import jax.numpy as jnp
from jax import lax
from jax.experimental import pallas as pl
from jax.experimental.pallas import tpu as pltpu
from jax.experimental.pallas import tpu_sc as plsc

_N = 10000
_E = 160000
_D = 128
_NW = 32
_CH = 128
_NCHUNK = _E // _CH
_CPW = _NCHUNK // _NW
_NEXTRA = _NCHUNK - _CPW * _NW
_WEXTRA = _NW - _NEXTRA
_PAIRS = (_CPW - 1) // 2
_ZR = 16
_NG = _N // _ZR


def _chunk_base(wid):
    return jnp.where(wid < _WEXTRA, _CPW * wid,
                     _CPW * _WEXTRA + (_CPW + 1) * (wid - _WEXTRA))


def _sc_mesh():
    return plsc.VectorSubcoreMesh(core_axis_name="c", subcore_axis_name="s")


def _gather_pairs(x, src3, dst3):
    @functools.partial(
        pl.kernel,
        out_type=(jax.ShapeDtypeStruct((_E, _D), jnp.float32),
                  jax.ShapeDtypeStruct((_E, _D), jnp.float32)),
        mesh=_sc_mesh(),
        scratch_types=[
            pltpu.VMEM((_CPW + 1, 1, _CH), jnp.int32),
            pltpu.VMEM((_CPW + 1, 1, _CH), jnp.int32),
            pltpu.VMEM((_CH, _D), jnp.float32),
            pltpu.VMEM((_CH, _D), jnp.float32),
            pltpu.VMEM((_CH, _D), jnp.float32),
            pltpu.VMEM((_CH, _D), jnp.float32),
            pltpu.SemaphoreType.DMA,
            pltpu.SemaphoreType.DMA,
            pltpu.SemaphoreType.DMA,
        ],
    )
    def k(x_hbm, src_hbm, dst_hbm, xs_hbm, xd_hbm,
          isv, idv, rs0, rd0, rs1, rd1, semg0, semg1, semw):
        cid = lax.axis_index("c")
        sid = lax.axis_index("s")
        wid = sid * 2 + cid
        cb = _chunk_base(wid)
        extra = wid >= _WEXTRA

        pltpu.sync_copy(src_hbm.at[pl.ds(cb, _CPW)], isv.at[pl.ds(0, _CPW)])
        pltpu.sync_copy(dst_hbm.at[pl.ds(cb, _CPW)], idv.at[pl.ds(0, _CPW)])

        @pl.when(extra)
        def _():
            pltpu.sync_copy(src_hbm.at[pl.ds(cb + _CPW, 1)],
                            isv.at[pl.ds(_CPW, 1)])
            pltpu.sync_copy(dst_hbm.at[pl.ds(cb + _CPW, 1)],
                            idv.at[pl.ds(_CPW, 1)])

        def one(t, rs, rd, semg):
            off = pl.multiple_of((cb + t) * _CH, 8)
            gs = pltpu.async_copy(x_hbm.at[isv.at[t, 0]], rs, semg)
            gd = pltpu.async_copy(x_hbm.at[idv.at[t, 0]], rd, semg)
            return gs, gd, off

        def pair(p, carry):
            t0 = p * 2
            gs0, gd0, off0 = one(t0, rs0, rd0, semg0)
            gs1, gd1, off1 = one(t0 + 1, rs1, rd1, semg1)
            gs0.wait()
            gd0.wait()
            w0 = pltpu.async_copy(rs0, xs_hbm.at[pl.ds(off0, _CH)], semw)
            w1 = pltpu.async_copy(rd0, xd_hbm.at[pl.ds(off0, _CH)], semw)
            gs1.wait()
            gd1.wait()
            w2 = pltpu.async_copy(rs1, xs_hbm.at[pl.ds(off1, _CH)], semw)
            w3 = pltpu.async_copy(rd1, xd_hbm.at[pl.ds(off1, _CH)], semw)
            w0.wait()
            w1.wait()
            w2.wait()
            w3.wait()
            return carry

        lax.fori_loop(0, _PAIRS, pair, 0)

        gs0, gd0, off0 = one(_CPW - 1, rs0, rd0, semg0)
        gs0.wait()
        gd0.wait()
        w0 = pltpu.async_copy(rs0, xs_hbm.at[pl.ds(off0, _CH)], semw)
        w1 = pltpu.async_copy(rd0, xd_hbm.at[pl.ds(off0, _CH)], semw)

        @pl.when(extra)
        def _():
            gs1, gd1, off1 = one(_CPW, rs1, rd1, semg1)
            gs1.wait()
            gd1.wait()
            w2 = pltpu.async_copy(rs1, xs_hbm.at[pl.ds(off1, _CH)], semw)
            w3 = pltpu.async_copy(rd1, xd_hbm.at[pl.ds(off1, _CH)], semw)
            w2.wait()
            w3.wait()

        w0.wait()
        w1.wait()

    return k(x, src3, dst3)


def _segment_partials(e, dst3):
    @functools.partial(
        pl.kernel,
        out_type=jax.ShapeDtypeStruct((2, _N, _D), jnp.float32),
        mesh=_sc_mesh(),
        scratch_types=[
            pltpu.VMEM((_CPW + 1, 1, _CH), jnp.int32),
            pltpu.VMEM((_CH, _D), jnp.float32),
            pltpu.VMEM((_CH, _D), jnp.float32),
            pltpu.VMEM((_ZR, _D), jnp.float32),
            pltpu.VMEM_SHARED((_N, _D), jnp.float32),
            pltpu.SemaphoreType.DMA,
            pltpu.SemaphoreType.DMA,
            pltpu.SemaphoreType.DMA,
        ],
    )
    def k(e_hbm, dst_hbm, out_hbm, idv, r0, r1, zbuf, acc, seml0, seml1, sems):
        cid = lax.axis_index("c")
        sid = lax.axis_index("s")
        wid = sid * 2 + cid
        cb = _chunk_base(wid)
        extra = wid >= _WEXTRA

        zero = jnp.zeros((16,), jnp.float32)
        for r in range(_ZR):
            for l in range(_D // 16):
                zbuf[r, pl.ds(16 * l, 16)] = zero
        ngrp = jnp.where(sid < _NG - 16 * (_NG // 16), _NG // 16 + 1, _NG // 16)

        def zfill(gk, carry):
            roff = pl.multiple_of((sid + gk * 16) * _ZR, 8)
            pltpu.sync_copy(zbuf, acc.at[pl.ds(roff, _ZR)])
            return carry

        lax.fori_loop(0, ngrp, zfill, 0)

        pltpu.sync_copy(dst_hbm.at[pl.ds(cb, _CPW)], idv.at[pl.ds(0, _CPW)])

        @pl.when(extra)
        def _():
            pltpu.sync_copy(dst_hbm.at[pl.ds(cb + _CPW, 1)],
                            idv.at[pl.ds(_CPW, 1)])

        plsc.subcore_barrier()

        def load(t, r, seml):
            off = pl.multiple_of((cb + t) * _CH, 8)
            return pltpu.async_copy(e_hbm.at[pl.ds(off, _CH)], r, seml)

        def pair(p, carry):
            t0 = p * 2
            l0 = load(t0, r0, seml0)
            l1 = load(t0 + 1, r1, seml1)
            l0.wait()
            s0 = pltpu.async_copy(r0, acc.at[idv.at[t0, 0]], sems, add=True)
            l1.wait()
            s1 = pltpu.async_copy(r1, acc.at[idv.at[t0 + 1, 0]], sems, add=True)
            s0.wait()
            s1.wait()
            return carry

        lax.fori_loop(0, _PAIRS, pair, 0)

        l0 = load(_CPW - 1, r0, seml0)
        l0.wait()
        s0 = pltpu.async_copy(r0, acc.at[idv.at[_CPW - 1, 0]], sems, add=True)

        @pl.when(extra)
        def _():
            l1 = load(_CPW, r1, seml1)
            l1.wait()
            s1 = pltpu.async_copy(r1, acc.at[idv.at[_CPW, 0]], sems, add=True)
            s1.wait()

        s0.wait()
        plsc.subcore_barrier()

        def wback(gk, carry):
            roff = pl.multiple_of((sid + gk * 16) * _ZR, 8)
            pltpu.sync_copy(acc.at[pl.ds(roff, _ZR)],
                            out_hbm.at[cid, pl.ds(roff, _ZR)])
            return carry

        lax.fori_loop(0, ngrp, wback, 0)

    return k(e, dst3)


def _mlp_pallas(groups, w1s, b1, w2, b2, w3, b3, ln_g, ln_b, resid_gi, br):
    flat = [a for g in groups for a in g]
    sizes = [len(g) for g in groups]
    n = flat[0].shape[0]
    num_in = len(flat)
    ng = len(w1s)
    has_ln = ln_g is not None

    row_spec = lambda d: pl.BlockSpec((br, d), lambda i: (i, 0))
    full_spec = lambda s: pl.BlockSpec(s, lambda i: (0, 0))

    in_specs = [row_spec(a.shape[1]) for a in flat]
    in_specs += [full_spec(w.shape) for w in w1s]
    b1r = b1.reshape(1, _D)
    b2r = b2.reshape(1, _D)
    b3r = b3.reshape(1, _D)
    extras = [b1r, w2, b2r, w3, b3r]
    in_specs += [full_spec(b1r.shape), full_spec(w2.shape),
                 full_spec(b2r.shape), full_spec(w3.shape),
                 full_spec(b3r.shape)]
    if has_ln:
        extras += [ln_g.reshape(1, _D), ln_b.reshape(1, _D)]
        in_specs += [full_spec((1, _D)), full_spec((1, _D))]

    def body(*refs):
        irefs = refs[:num_in]
        wrefs = refs[num_in:num_in + ng]
        rest = refs[num_in + ng:]
        b1_r, w2_r, b2_r, w3_r, b3_r = rest[:5]
        out_ref = refs[-1]
        pos = 0
        h = None
        res = None
        for gi, sz in enumerate(sizes):
            xg = irefs[pos][...]
            for j in range(1, sz):
                xg = xg + irefs[pos + j][...]
            t = jnp.dot(xg, wrefs[gi][...], preferred_element_type=jnp.float32)
            h = t if h is None else h + t
            if resid_gi is not None and gi == resid_gi:
                res = xg
            pos += sz
        h = jax.nn.relu(h + b1_r[...])
        h = jax.nn.relu(jnp.dot(h, w2_r[...], preferred_element_type=jnp.float32) + b2_r[...])
        h = jnp.dot(h, w3_r[...], preferred_element_type=jnp.float32) + b3_r[...]
        if has_ln:
            g_r, bb_r = rest[5], rest[6]
            mu = jnp.mean(h, axis=-1, keepdims=True)
            var = jnp.mean((h - mu) * (h - mu), axis=-1, keepdims=True)
            h = (h - mu) * lax.rsqrt(var + 1e-5) * g_r[...] + bb_r[...]
        if res is not None:
            h = res + h
        out_ref[...] = h

    return pl.pallas_call(
        body,
        grid=(n // br,),
        in_specs=in_specs,
        out_specs=pl.BlockSpec((br, _D), lambda i: (i, 0)),
        out_shape=jax.ShapeDtypeStruct((n, _D), jnp.float32),
    )(*flat, *w1s, *extras)


def kernel(init_position, time_vector, node_type, edge_index, edge_features, params):
    p = params
    onehot = jax.nn.one_hot(node_type, 9, dtype=jnp.float32)
    nf = jnp.concatenate(
        [init_position, time_vector[:, None], onehot,
         jnp.zeros((_N, 4), jnp.float32)], axis=1)
    ef = jnp.concatenate(
        [edge_features, jnp.zeros((_E, 5), jnp.float32)], axis=1)
    src3 = edge_index[0].reshape(_NCHUNK, 1, _CH)
    dst3 = edge_index[1].reshape(_NCHUNK, 1, _CH)

    ne = p["node_enc"]
    w1n = jnp.concatenate([ne["W"][0], jnp.zeros((4, _D), jnp.float32)], axis=0)
    x = _mlp_pallas([[nf]], [w1n], ne["b"][0], ne["W"][1], ne["b"][1],
                    ne["W"][2], ne["b"][2], ne["ln_g"], ne["ln_b"],
                    resid_gi=None, br=1000)

    ee = p["edge_enc"]
    w1e = jnp.concatenate([ee["W"][0], jnp.zeros((5, _D), jnp.float32)], axis=0)
    e = _mlp_pallas([[ef]], [w1e], ee["b"][0], ee["W"][1], ee["b"][1],
                    ee["W"][2], ee["b"][2], ee["ln_g"], ee["ln_b"],
                    resid_gi=None, br=1000)

    for sp in p["proc"]:
        xs, xd = _gather_pairs(x, src3, dst3)
        ew = sp["edge"]
        wa = ew["W"][0][:_D]
        wb = ew["W"][0][_D:2 * _D]
        wc = ew["W"][0][2 * _D:]
        e = _mlp_pallas([[xs], [xd], [e]], [wa, wb, wc], ew["b"][0],
                        ew["W"][1], ew["b"][1], ew["W"][2], ew["b"][2],
                        ew["ln_g"], ew["ln_b"], resid_gi=2, br=1000)
        parts = _segment_partials(e, dst3)
        nw = sp["node"]
        na = nw["W"][0][:_D]
        nb = nw["W"][0][_D:]
        x = _mlp_pallas([[x], [parts[0], parts[1]]], [na, nb], nw["b"][0],
                        nw["W"][1], nw["b"][1], nw["W"][2], nw["b"][2],
                        nw["ln_g"], nw["ln_b"], resid_gi=0, br=1000)

    dp = p["dec"]
    w3d = jnp.pad(dp["W"][2], ((0, 0), (0, _D - 2)))
    b3d = jnp.pad(dp["b"][2], (0, _D - 2))
    out = _mlp_pallas([[x]], [dp["W"][0]], dp["b"][0], dp["W"][1], dp["b"][1],
                      w3d, b3d, None, None, resid_gi=None, br=1000)
    return init_position + out[:, :2]
